# baseline (speedup 1.0000x reference)
import functools

import jax
import jax.numpy as jnp
from jax import lax
from jax.experimental import pallas as pl
from jax.experimental.pallas import tpu as pltpu
from jax.experimental.pallas import tpu_sc as plsc

D_MODEL = 512
FP_DIM = 2048
N_NODES = 16384
N_GRAPHS = 16
NEG = -1e30

NW = 32
N_SC = 4096
N_TC = N_NODES - N_SC
NPW = N_SC // NW
BLK = 4096
NBLK_TC = N_TC // BLK

_mesh = plsc.VectorSubcoreMesh(core_axis_name="c", subcore_axis_name="s")


def _prologue_body(kfeat_ref, wq_ref, wk_ref, kf_ref, proj_ref):
    kf = lax.dot_general(kfeat_ref[...], wq_ref[...], (((1,), (1,)), ((), ())),
                         preferred_element_type=jnp.float32)
    kf_ref[...] = kf
    proj_ref[...] = jnp.dot(kf, wk_ref[...],
                            preferred_element_type=jnp.float32) * (
                                1.0 / (D_MODEL ** 0.5))


def _prologue(k_feature, W_q, W_k):
    return pl.pallas_call(
        _prologue_body,
        out_shape=[jax.ShapeDtypeStruct((N_GRAPHS, D_MODEL), jnp.float32),
                   jax.ShapeDtypeStruct((N_GRAPHS, D_MODEL), jnp.float32)],
    )(k_feature, W_q, W_k)


@functools.partial(
    pl.kernel,
    out_type=[jax.ShapeDtypeStruct((NW, N_GRAPHS), jnp.float32),
              jax.ShapeDtypeStruct((NW, N_GRAPHS), jnp.float32),
              jax.ShapeDtypeStruct((NW, N_GRAPHS * D_MODEL), jnp.float32)],
    mesh=_mesh,
    scratch_types=[
        pltpu.VMEM((N_GRAPHS * D_MODEL,), jnp.float32),
        pltpu.VMEM((NPW,), jnp.int32),
        pltpu.VMEM((NPW * D_MODEL,), jnp.float32),
        pltpu.VMEM((NPW,), jnp.float32),
        pltpu.VMEM((N_GRAPHS,), jnp.float32),
        pltpu.VMEM((N_GRAPHS,), jnp.float32),
        pltpu.VMEM((N_GRAPHS * D_MODEL,), jnp.float32),
    ],
    compiler_params=pltpu.CompilerParams(use_tc_tiling_on_sc=False,
                                         needs_layout_passes=False),
)
def _sc_partial(x_hbm, seg_hbm, proj_hbm, m_out, s_out, acc_out,
                proj_v, seg_v, chunk_v, score_v, m_v, s_v, acc_v):
    wid = lax.axis_index("s") * 2 + lax.axis_index("c")
    node0 = N_TC + wid * NPW

    pltpu.sync_copy(proj_hbm, proj_v)
    pltpu.sync_copy(seg_hbm.at[pl.ds(node0, NPW)], seg_v)
    pltpu.sync_copy(x_hbm.at[pl.ds(node0 * D_MODEL, NPW * D_MODEL)], chunk_v)

    giota = lax.broadcasted_iota(jnp.int32, (N_GRAPHS,), 0)
    zero16 = jnp.zeros((16,), jnp.float32)

    def _zero(j, _):
        acc_v[pl.ds(j * 16, 16)] = zero16
        return 0
    lax.fori_loop(0, (N_GRAPHS * D_MODEL) // 16, _zero, 0)

    def _score_grp(grp, m16):
        n0 = grp * 16
        seg16 = seg_v[pl.ds(n0, 16)]
        xbase = (lax.broadcasted_iota(jnp.int32, (16,), 0) + n0) * D_MODEL
        pbase = seg16 * D_MODEL

        def _dim8(jj, sc16):
            for u in range(8):
                col = jnp.full((16,), jj * 8 + u, jnp.int32)
                xcol = plsc.load_gather(chunk_v, [xbase + col])
                pcol = plsc.load_gather(proj_v, [pbase + col])
                sc16 = sc16 + xcol * pcol
            return sc16

        sc16 = lax.fori_loop(0, D_MODEL // 8, _dim8, zero16)
        score_v[pl.ds(n0, 16)] = sc16
        for lane in range(16):
            g = seg16[lane]
            sc = sc16[lane]
            m16 = jnp.where(giota == g, jnp.maximum(m16, sc), m16)
        return m16

    m16 = lax.fori_loop(0, NPW // 16, _score_grp,
                        jnp.full((N_GRAPHS,), NEG, jnp.float32))
    m_v[...] = m16

    def _acc_grp(grp, s16):
        n0 = grp * 16
        seg16 = seg_v[pl.ds(n0, 16)]
        sc16 = score_v[pl.ds(n0, 16)]
        mg16 = plsc.load_gather(m_v, [seg16])
        e16 = jnp.exp(sc16 - mg16)
        for lane in range(16):
            n = n0 + lane
            g = seg16[lane]
            w = e16[lane]
            s16 = s16 + jnp.where(giota == g, w, 0.0)
            row = g * D_MODEL
            xrow = n * D_MODEL
            for j in range(D_MODEL // 16):
                off = row + j * 16
                acc_v[pl.ds(off, 16)] = (
                    acc_v[pl.ds(off, 16)]
                    + w * chunk_v[pl.ds(xrow + j * 16, 16)])
        return s16

    s16 = lax.fori_loop(0, NPW // 16, _acc_grp,
                        jnp.zeros((N_GRAPHS,), jnp.float32))
    s_v[...] = s16

    pltpu.sync_copy(m_v, m_out.at[wid])
    pltpu.sync_copy(s_v, s_out.at[wid])
    pltpu.sync_copy(acc_v, acc_out.at[wid])


def _main_body(seg_ref, x_ref, proj_ref, mo_ref, so_ref, acco_ref,
               m_ref, s_ref, acc_ref):
    i = pl.program_id(0)

    @pl.when(i == 0)
    def _init():
        m_ref[...] = jnp.full((N_GRAPHS, 1), NEG, jnp.float32)
        s_ref[...] = jnp.zeros((N_GRAPHS, 1), jnp.float32)
        acc_ref[...] = jnp.zeros((N_GRAPHS, D_MODEL), jnp.float32)

    x = x_ref[...]
    seg = seg_ref[0]
    st = lax.dot_general(proj_ref[...], x, (((1,), (1,)), ((), ())),
                         preferred_element_type=jnp.float32)
    gids = lax.broadcasted_iota(jnp.int32, (N_GRAPHS, BLK), 0)
    mask = jnp.broadcast_to(seg, (N_GRAPHS, BLK)) == gids
    sm = jnp.where(mask, st, NEG)
    bm = jnp.max(sm, axis=1, keepdims=True)
    m_old = m_ref[...]
    m_new = jnp.maximum(m_old, bm)
    scale = jnp.exp(m_old - m_new)
    e = jnp.where(mask, jnp.exp(st - m_new), 0.0)
    s_ref[...] = s_ref[...] * scale + jnp.sum(e, axis=1, keepdims=True)
    m_ref[...] = m_new
    acc_ref[...] = acc_ref[...] * scale + jnp.dot(
        e, x, preferred_element_type=jnp.float32)

    @pl.when(i == NBLK_TC - 1)
    def _emit():
        mo_ref[...] = m_ref[...]
        so_ref[...] = s_ref[...]
        acco_ref[...] = acc_ref[...]


def _main(seg3, node_feature, proj):
    fixed = lambda i: (0, 0)
    return pl.pallas_call(
        _main_body,
        grid=(NBLK_TC,),
        in_specs=[
            pl.BlockSpec((1, 1, BLK), lambda i: (i, 0, 0)),
            pl.BlockSpec((BLK, D_MODEL), lambda i: (i, 0)),
            pl.BlockSpec((N_GRAPHS, D_MODEL), fixed),
        ],
        out_specs=[
            pl.BlockSpec((N_GRAPHS, 1), fixed),
            pl.BlockSpec((N_GRAPHS, 1), fixed),
            pl.BlockSpec((N_GRAPHS, D_MODEL), fixed),
        ],
        out_shape=[jax.ShapeDtypeStruct((N_GRAPHS, 1), jnp.float32),
                   jax.ShapeDtypeStruct((N_GRAPHS, 1), jnp.float32),
                   jax.ShapeDtypeStruct((N_GRAPHS, D_MODEL), jnp.float32)],
        scratch_shapes=[
            pltpu.VMEM((N_GRAPHS, 1), jnp.float32),
            pltpu.VMEM((N_GRAPHS, 1), jnp.float32),
            pltpu.VMEM((N_GRAPHS, D_MODEL), jnp.float32),
        ],
        compiler_params=pltpu.CompilerParams(
            dimension_semantics=("arbitrary",)),
    )(seg3, node_feature, proj)


def _merge_body(mtc_ref, stc_ref, acctc_ref, msc_ref, ssc_ref, accsc_ref,
                kf_ref, wv_ref, bv_ref, wih_ref, whh_ref, bih_ref, bhh_ref,
                out_ref):
    eye = (lax.broadcasted_iota(jnp.int32, (N_GRAPHS, N_GRAPHS), 0) ==
           lax.broadcasted_iota(jnp.int32, (N_GRAPHS, N_GRAPHS), 1)
           ).astype(jnp.float32)

    def col_of(row):
        return lax.dot_general(eye, row, (((1,), (1,)), ((), ())),
                               preferred_element_type=jnp.float32)

    def row_of(col):
        return lax.dot_general(col, eye, (((0,), (0,)), ((), ())),
                               preferred_element_type=jnp.float32)

    m_tc = mtc_ref[...]
    msc = msc_ref[...]
    m_sc_row = jnp.max(msc, axis=0, keepdims=True)
    m_col = jnp.maximum(m_tc, col_of(m_sc_row))
    m_row = row_of(m_col)

    ftc = jnp.exp(m_tc - m_col)
    fsc = jnp.exp(msc - m_row)

    s_col = stc_ref[...] * ftc + col_of(
        jnp.sum(ssc_ref[...] * fsc, axis=0, keepdims=True))
    acc = acctc_ref[...] * ftc + lax.dot_general(
        fsc, accsc_ref[...], (((0,), (0,)), ((1,), (1,))),
        preferred_element_type=jnp.float32)

    has = (s_col > 0.0).astype(jnp.float32)
    pooled = acc / jnp.where(s_col > 0.0, s_col, 1.0)
    out = lax.dot_general(pooled, wv_ref[...], (((1,), (1,)), ((), ())),
                          preferred_element_type=jnp.float32)
    out = out + has * bv_ref[...]
    kf = kf_ref[...]
    gi = lax.dot_general(out, wih_ref[...], (((1,), (1,)), ((), ())),
                         preferred_element_type=jnp.float32) + bih_ref[...]
    gh = lax.dot_general(kf, whh_ref[...], (((1,), (1,)), ((), ())),
                         preferred_element_type=jnp.float32) + bhh_ref[...]
    i_r = gi[:, :D_MODEL]
    i_z = gi[:, D_MODEL:2 * D_MODEL]
    i_n = gi[:, 2 * D_MODEL:]
    h_r = gh[:, :D_MODEL]
    h_z = gh[:, D_MODEL:2 * D_MODEL]
    h_n = gh[:, 2 * D_MODEL:]
    r = jax.nn.sigmoid(i_r + h_r)
    z = jax.nn.sigmoid(i_z + h_z)
    n = jnp.tanh(i_n + r * h_n)
    out_ref[...] = (1.0 - z) * n + z * kf


def _merge(m_tc, s_tc, acc_tc, m_sc, s_sc, acc_sc3, kf,
           W_v, bv2, W_ih, W_hh, bih2, bhh2):
    return pl.pallas_call(
        _merge_body,
        out_shape=jax.ShapeDtypeStruct((N_GRAPHS, D_MODEL), jnp.float32),
    )(m_tc, s_tc, acc_tc, m_sc, s_sc, acc_sc3, kf,
      W_v, bv2, W_ih, W_hh, bih2, bhh2)


@jax.jit
def kernel(node_feature, k_feature, segment_ids, W_q, W_k, W_v, b_v,
           W_ih, W_hh, b_ih, b_hh):
    seg_i32 = segment_ids.astype(jnp.int32)
    seg3 = seg_i32.reshape(N_NODES // BLK, 1, BLK)
    bv2 = b_v.reshape(1, D_MODEL)
    bih2 = b_ih.reshape(1, 3 * D_MODEL)
    bhh2 = b_hh.reshape(1, 3 * D_MODEL)

    kf, proj = _prologue(k_feature, W_q, W_k)
    m_sc, s_sc, acc_sc = _sc_partial(node_feature.reshape(-1), seg_i32,
                                     proj.reshape(-1))
    m_tc, s_tc, acc_tc = _main(seg3, node_feature, proj)
    acc_sc3 = acc_sc.reshape(NW, N_GRAPHS, D_MODEL)
    return _merge(m_tc, s_tc, acc_tc, m_sc, s_sc, acc_sc3, kf,
                  W_v, bv2, W_ih, W_hh, bih2, bhh2)


# speedup vs baseline: 6.8375x; 6.8375x over previous
import jax
import jax.numpy as jnp
from jax import lax
from jax.experimental import pallas as pl
from jax.experimental.pallas import tpu as pltpu

D_MODEL = 512
FP_DIM = 2048
N_NODES = 16384
N_GRAPHS = 16
BLK = 8192
NBLK = N_NODES // BLK
NEG = -1e30


def _fused(seg_ref, x_ref, kfeat_ref, wq_ref, wk_ref, bv_ref,
           bih_ref, bhh_ref, wv_hbm, wih_hbm, whh_hbm,
           out_ref, kf_ref, proj_ref, m_ref, s_ref, acc_ref,
           wv_s, wih_s, whh_s, sem_v, sem_ih, sem_hh):
    i = pl.program_id(0)

    @pl.when(i == 0)
    def _prologue():
        pltpu.make_async_copy(wv_hbm, wv_s, sem_v).start()
        pltpu.make_async_copy(wih_hbm, wih_s, sem_ih).start()
        pltpu.make_async_copy(whh_hbm, whh_s, sem_hh).start()
        kf = lax.dot_general(kfeat_ref[...], wq_ref[...],
                             (((1,), (1,)), ((), ())),
                             preferred_element_type=jnp.float32)
        kf_ref[...] = kf
        proj_ref[...] = jnp.dot(kf, wk_ref[...],
                                preferred_element_type=jnp.float32) * (
                                    1.0 / (D_MODEL ** 0.5))
        m_ref[...] = jnp.full((N_GRAPHS, 1), NEG, jnp.float32)
        s_ref[...] = jnp.zeros((N_GRAPHS, 1), jnp.float32)
        acc_ref[...] = jnp.zeros((N_GRAPHS, D_MODEL), jnp.float32)

    x = x_ref[...]
    seg = seg_ref[0]
    st = lax.dot_general(proj_ref[...], x, (((1,), (1,)), ((), ())),
                         preferred_element_type=jnp.float32)
    gids = lax.broadcasted_iota(jnp.int32, (N_GRAPHS, BLK), 0)
    mask = jnp.broadcast_to(seg, (N_GRAPHS, BLK)) == gids
    sm = jnp.where(mask, st, NEG)
    bm = jnp.max(sm, axis=1, keepdims=True)
    m_old = m_ref[...]
    m_new = jnp.maximum(m_old, bm)
    scale = jnp.exp(m_old - m_new)
    e = jnp.where(mask, jnp.exp(st - m_new), 0.0)
    s_ref[...] = s_ref[...] * scale + jnp.sum(e, axis=1, keepdims=True)
    m_ref[...] = m_new
    acc_ref[...] = acc_ref[...] * scale + jnp.dot(
        e, x, preferred_element_type=jnp.float32)

    @pl.when(i == NBLK - 1)
    def _epilogue():
        pltpu.make_async_copy(wv_hbm, wv_s, sem_v).wait()
        pltpu.make_async_copy(wih_hbm, wih_s, sem_ih).wait()
        pltpu.make_async_copy(whh_hbm, whh_s, sem_hh).wait()
        s = s_ref[...]
        has = (s > 0.0).astype(jnp.float32)
        pooled = acc_ref[...] / jnp.where(s > 0.0, s, 1.0)
        out = lax.dot_general(pooled, wv_s[...], (((1,), (1,)), ((), ())),
                              preferred_element_type=jnp.float32)
        out = out + has * bv_ref[...]
        kf = kf_ref[...]
        gi = lax.dot_general(out, wih_s[...], (((1,), (1,)), ((), ())),
                             preferred_element_type=jnp.float32) + bih_ref[...]
        gh = lax.dot_general(kf, whh_s[...], (((1,), (1,)), ((), ())),
                             preferred_element_type=jnp.float32) + bhh_ref[...]
        i_r = gi[:, :D_MODEL]
        i_z = gi[:, D_MODEL:2 * D_MODEL]
        i_n = gi[:, 2 * D_MODEL:]
        h_r = gh[:, :D_MODEL]
        h_z = gh[:, D_MODEL:2 * D_MODEL]
        h_n = gh[:, 2 * D_MODEL:]
        r = jax.nn.sigmoid(i_r + h_r)
        z = jax.nn.sigmoid(i_z + h_z)
        n = jnp.tanh(i_n + r * h_n)
        out_ref[...] = (1.0 - z) * n + z * kf


@jax.jit
def kernel(node_feature, k_feature, segment_ids, W_q, W_k, W_v, b_v,
           W_ih, W_hh, b_ih, b_hh):
    seg = segment_ids.astype(jnp.int32).reshape(NBLK, 1, BLK)
    bv2 = b_v.reshape(1, D_MODEL)
    bih2 = b_ih.reshape(1, 3 * D_MODEL)
    bhh2 = b_hh.reshape(1, 3 * D_MODEL)

    fixed = lambda i: (0, 0)
    out = pl.pallas_call(
        _fused,
        grid=(NBLK,),
        in_specs=[
            pl.BlockSpec((1, 1, BLK), lambda i: (i, 0, 0)),
            pl.BlockSpec((BLK, D_MODEL), lambda i: (i, 0)),
            pl.BlockSpec((N_GRAPHS, FP_DIM), fixed),
            pl.BlockSpec((D_MODEL, FP_DIM), fixed),
            pl.BlockSpec((D_MODEL, D_MODEL), fixed),
            pl.BlockSpec((1, D_MODEL), fixed),
            pl.BlockSpec((1, 3 * D_MODEL), fixed),
            pl.BlockSpec((1, 3 * D_MODEL), fixed),
            pl.BlockSpec(memory_space=pltpu.HBM),
            pl.BlockSpec(memory_space=pltpu.HBM),
            pl.BlockSpec(memory_space=pltpu.HBM),
        ],
        out_specs=pl.BlockSpec((N_GRAPHS, D_MODEL), fixed),
        out_shape=jax.ShapeDtypeStruct((N_GRAPHS, D_MODEL), jnp.float32),
        scratch_shapes=[
            pltpu.VMEM((N_GRAPHS, D_MODEL), jnp.float32),
            pltpu.VMEM((N_GRAPHS, D_MODEL), jnp.float32),
            pltpu.VMEM((N_GRAPHS, 1), jnp.float32),
            pltpu.VMEM((N_GRAPHS, 1), jnp.float32),
            pltpu.VMEM((N_GRAPHS, D_MODEL), jnp.float32),
            pltpu.VMEM((D_MODEL, D_MODEL), jnp.float32),
            pltpu.VMEM((3 * D_MODEL, D_MODEL), jnp.float32),
            pltpu.VMEM((3 * D_MODEL, D_MODEL), jnp.float32),
            pltpu.SemaphoreType.DMA,
            pltpu.SemaphoreType.DMA,
            pltpu.SemaphoreType.DMA,
        ],
        compiler_params=pltpu.CompilerParams(
            dimension_semantics=("arbitrary",)),
    )(seg, node_feature, k_feature, W_q, W_k, bv2, bih2, bhh2,
      W_v, W_ih, W_hh)
    return out
